# Initial kernel scaffold; baseline (speedup 1.0000x reference)
import jax
import jax.numpy as jnp
from jax.experimental import pallas as pl


def kernel(codes, codebooks):
    raise NotImplementedError("write your pallas kernel here")


# speedup vs baseline: 3.2160x; 3.2160x over previous
import functools

import jax
import jax.numpy as jnp
from jax import lax
from jax.experimental import pallas as pl
from jax.experimental.pallas import tpu as pltpu
from jax.experimental.pallas import tpu_sc as plsc

N_CB = 8
VOCAB = 1024
D = 128
B = 16
T = 2048
L = 16
NC, NS = 2, 16
NW = NC * NS
NG = 16
D_PER_G = D // NG
B_PER_H = B // 2
TC_CHUNKS = T // L

_mesh = plsc.VectorSubcoreMesh(
    core_axis_name="c", subcore_axis_name="s", num_cores=NC, num_subcores=NS
)


@functools.partial(
    pl.kernel,
    out_type=jax.ShapeDtypeStruct((B, N_CB * D, T), jnp.float32),
    mesh=_mesh,
    compiler_params=pltpu.CompilerParams(needs_layout_passes=False),
    scratch_types=[
        pltpu.VMEM((N_CB, D_PER_G, VOCAB), jnp.float32),
        pltpu.VMEM((N_CB, T), jnp.int32),
        pltpu.VMEM((2, D_PER_G, T), jnp.float32),
        pltpu.SemaphoreType.DMA,
        pltpu.SemaphoreType.DMA,
    ],
)
def _codes_to_features(cbt_hbm, codes_hbm, out_hbm, cbk_v, codes_v, obuf_v, sem0, sem1):
    wid = lax.axis_index("s") * NC + lax.axis_index("c")
    g = wid % NG
    h = wid // NG
    sems = (sem0, sem1)

    pltpu.sync_copy(cbt_hbm.at[:, g], cbk_v)

    @pl.loop(0, B_PER_H)
    def _b_loop(bi):
        b = h * B_PER_H + bi
        pltpu.sync_copy(codes_hbm.at[b], codes_v)
        descs = [None, None]
        for cb in range(N_CB):
            p = cb & 1
            if descs[p] is not None:
                descs[p].wait()

            @pl.loop(0, TC_CHUNKS)
            def _tc_loop(tc):
                idx = codes_v[cb, pl.ds(tc * L, L)]
                cb_i = jnp.full((L,), cb, jnp.int32)
                for dl in range(D_PER_G):
                    dl_i = jnp.full((L,), dl, jnp.int32)
                    row = plsc.load_gather(cbk_v, [cb_i, dl_i, idx])
                    obuf_v[p, dl, pl.ds(tc * L, L)] = row

            row0 = pl.multiple_of(cb * D + g * D_PER_G, D_PER_G)
            descs[p] = pltpu.async_copy(
                obuf_v.at[p],
                out_hbm.at[b, pl.ds(row0, D_PER_G), :],
                sems[p],
            )
        descs[0].wait()
        descs[1].wait()


def kernel(codes, codebooks):
    cbt = jnp.swapaxes(codebooks, 1, 2).reshape(N_CB, NG, D_PER_G, VOCAB)
    return _codes_to_features(cbt, codes)


# speedup vs baseline: 10.3147x; 3.2073x over previous
import functools

import jax
import jax.numpy as jnp
from jax import lax
from jax.experimental import pallas as pl
from jax.experimental.pallas import tpu as pltpu
from jax.experimental.pallas import tpu_sc as plsc

N_CB = 8
VOCAB = 1024
D = 128
B = 16
T = 2048
L = 16
NC, NS = 2, 16
NW = NC * NS
NG = 16
D_PER_G = D // NG
B_PER_H = B // 2
TC_CHUNKS = T // L

_mesh = plsc.VectorSubcoreMesh(
    core_axis_name="c", subcore_axis_name="s", num_cores=NC, num_subcores=NS
)


@functools.partial(
    pl.kernel,
    out_type=jax.ShapeDtypeStruct((B, N_CB * D, T), jnp.float32),
    mesh=_mesh,
    compiler_params=pltpu.CompilerParams(needs_layout_passes=False),
    scratch_types=[
        pltpu.VMEM((N_CB, D_PER_G, VOCAB), jnp.float32),
        pltpu.VMEM((N_CB, T), jnp.int32),
        pltpu.VMEM((2, D_PER_G, T), jnp.float32),
        pltpu.SemaphoreType.DMA,
        pltpu.SemaphoreType.DMA,
    ],
)
def _codes_to_features(cbt_hbm, codes_hbm, out_hbm, cbk_v, codes_v, obuf_v, sem0, sem1):
    wid = lax.axis_index("s") * NC + lax.axis_index("c")
    g = wid % NG
    h = wid // NG
    sems = (sem0, sem1)

    pltpu.sync_copy(cbt_hbm.at[:, g], cbk_v)

    @pl.loop(0, B_PER_H)
    def _b_loop(bi):
        b = h * B_PER_H + bi
        pltpu.sync_copy(codes_hbm.at[b], codes_v)
        descs = [None, None]
        for cb in range(N_CB):
            p = cb & 1
            if descs[p] is not None:
                descs[p].wait()

            @plsc.parallel_loop(0, T, step=L, unroll=8)
            def _tc_loop(t0):
                idx = codes_v[cb, pl.ds(t0, L)]
                cb_i = jnp.full((L,), cb, jnp.int32)
                for dl in range(D_PER_G):
                    dl_i = jnp.full((L,), dl, jnp.int32)
                    row = plsc.load_gather(cbk_v, [cb_i, dl_i, idx])
                    obuf_v[p, dl, pl.ds(t0, L)] = row

            row0 = pl.multiple_of(cb * D + g * D_PER_G, D_PER_G)
            descs[p] = pltpu.async_copy(
                obuf_v.at[p],
                out_hbm.at[b, pl.ds(row0, D_PER_G), :],
                sems[p],
            )
        descs[0].wait()
        descs[1].wait()


def kernel(codes, codebooks):
    cbt = jnp.swapaxes(codebooks, 1, 2).reshape(N_CB, NG, D_PER_G, VOCAB)
    return _codes_to_features(cbt, codes)


# speedup vs baseline: 12.0707x; 1.1702x over previous
import functools

import jax
import jax.numpy as jnp
from jax import lax
from jax.experimental import pallas as pl
from jax.experimental.pallas import tpu as pltpu
from jax.experimental.pallas import tpu_sc as plsc

N_CB = 8
VOCAB = 1024
D = 128
B = 16
T = 2048
L = 16
NC, NS = 2, 16
NW = NC * NS
NG = 16
D_PER_G = D // NG
B_PER_H = B // 2
TC_CHUNKS = T // L

_mesh = plsc.VectorSubcoreMesh(
    core_axis_name="c", subcore_axis_name="s", num_cores=NC, num_subcores=NS
)


@functools.partial(
    pl.kernel,
    out_type=jax.ShapeDtypeStruct((B, N_CB * D, T), jnp.float32),
    mesh=_mesh,
    compiler_params=pltpu.CompilerParams(needs_layout_passes=False),
    scratch_types=[
        pltpu.VMEM((N_CB, D_PER_G, VOCAB), jnp.float32),
        pltpu.VMEM((2, N_CB, T), jnp.int32),
        pltpu.VMEM((2, D_PER_G, T), jnp.float32),
        pltpu.SemaphoreType.DMA,
        pltpu.SemaphoreType.DMA,
        pltpu.SemaphoreType.DMA,
        pltpu.SemaphoreType.DMA,
    ],
)
def _codes_to_features(
    cbt_hbm, codes_hbm, out_hbm, cbk_v, codes_v, obuf_v, sem0, sem1, csem0, csem1
):
    wid = lax.axis_index("s") * NC + lax.axis_index("c")
    g = wid % NG
    h = wid // NG
    sems = (sem0, sem1)
    csems = (csem0, csem1)
    b_base = h * B_PER_H

    pltpu.async_copy(codes_hbm.at[b_base], codes_v.at[0], csems[0])
    pltpu.sync_copy(cbt_hbm.at[:, g], cbk_v)

    @pl.loop(0, B_PER_H, step=2)
    def _b_loop(bi0):
        descs = [None, None]
        for j in range(2):
            bi = bi0 + j
            b = b_base + bi
            pltpu.make_async_copy(codes_hbm.at[b], codes_v.at[j], csems[j]).wait()

            @pl.when(bi + 1 < B_PER_H)
            def _prefetch():
                pltpu.async_copy(
                    codes_hbm.at[b + 1], codes_v.at[j ^ 1], csems[j ^ 1]
                )

            for cb in range(N_CB):
                p = cb & 1
                if descs[p] is not None:
                    descs[p].wait()

                @plsc.parallel_loop(0, T, step=L, unroll=8)
                def _tc_loop(t0):
                    idx = codes_v[j, cb, pl.ds(t0, L)]
                    cb_i = jnp.full((L,), cb, jnp.int32)
                    for dl in range(D_PER_G):
                        dl_i = jnp.full((L,), dl, jnp.int32)
                        row = plsc.load_gather(cbk_v, [cb_i, dl_i, idx])
                        obuf_v[p, dl, pl.ds(t0, L)] = row

                row0 = pl.multiple_of(cb * D + g * D_PER_G, D_PER_G)
                descs[p] = pltpu.async_copy(
                    obuf_v.at[p],
                    out_hbm.at[b, pl.ds(row0, D_PER_G), :],
                    sems[p],
                )
        descs[0].wait()
        descs[1].wait()


def kernel(codes, codebooks):
    cbt = jnp.swapaxes(codebooks, 1, 2).reshape(N_CB, NG, D_PER_G, VOCAB)
    return _codes_to_features(cbt, codes)
